# baseline (speedup 1.0000x reference)
import jax
import jax.numpy as jnp
from jax.experimental import pallas as pl
from jax.experimental.pallas import tpu as pltpu

B = 16384
C = 10
NB = 2
RB = B // NB
GB = 128 // NB


def _body(x_hbm, g_hbm, soft_hbm, ohe_hbm,
          ebuf, xbuf, gbuf, obuf, sbuf, xsem, osem, ssem, gsem):
    def xcopy(b, slot):
        return pltpu.make_async_copy(
            x_hbm.at[pl.ds(b * RB, RB), :], xbuf.at[slot], xsem.at[slot])

    def ocopy(b, slot):
        return pltpu.make_async_copy(
            obuf.at[slot], ohe_hbm.at[pl.ds(b * GB, GB)], osem.at[slot])

    def scopy(b, slot):
        return pltpu.make_async_copy(
            sbuf.at[slot], soft_hbm.at[pl.ds(b * RB, RB), :], ssem.at[slot])

    gcopy = pltpu.make_async_copy(g_hbm, gbuf, gsem)
    gcopy.start()
    xcopy(0, 0).start()
    xcopy(1, 1).start()
    gcopy.wait()

    def pass1(b, acc):
        slot = jax.lax.rem(b, 2)

        @pl.when(b >= 2)
        def _drain():
            ocopy(b - 2, slot).wait()

        g = gbuf[pl.ds(b * GB, GB), :]
        cls = jax.lax.broadcasted_iota(jnp.int32, (GB, 128, C), 2)
        obuf[slot] = (g[:, :, None] == cls).astype(jnp.float32)
        ocopy(b, slot).start()

        xcopy(b, slot).wait()
        e = jnp.exp(xbuf[slot])
        ebuf[pl.ds(b * RB, RB), :] = e

        @pl.when(b + 2 < NB)
        def _prefetch():
            xcopy(b + 2, slot).start()

        return acc + jnp.sum(e)

    total = jax.lax.fori_loop(0, NB, pass1, 0.0)
    inv = 1.0 / total

    def pass2(b, carry):
        slot = jax.lax.rem(b, 2)

        @pl.when(b >= 2)
        def _drain():
            scopy(b - 2, slot).wait()

        sbuf[slot] = ebuf[pl.ds(b * RB, RB), :] * inv
        scopy(b, slot).start()
        return carry

    jax.lax.fori_loop(0, NB, pass2, 0)

    ocopy(NB - 2, 0).wait()
    ocopy(NB - 1, 1).wait()
    scopy(NB - 2, 0).wait()
    scopy(NB - 1, 1).wait()


def kernel(inference, ground_truth):
    gt128 = ground_truth.astype(jnp.int32).reshape(128, 128)
    soft, ohe3 = pl.pallas_call(
        _body,
        in_specs=[
            pl.BlockSpec(memory_space=pltpu.MemorySpace.HBM),
            pl.BlockSpec(memory_space=pltpu.MemorySpace.HBM),
        ],
        out_specs=[
            pl.BlockSpec(memory_space=pltpu.MemorySpace.HBM),
            pl.BlockSpec(memory_space=pltpu.MemorySpace.HBM),
        ],
        out_shape=(
            jax.ShapeDtypeStruct((B, C), jnp.float32),
            jax.ShapeDtypeStruct((128, 128, C), jnp.float32),
        ),
        scratch_shapes=[
            pltpu.VMEM((B, C), jnp.float32),
            pltpu.VMEM((2, RB, C), jnp.float32),
            pltpu.VMEM((128, 128), jnp.int32),
            pltpu.VMEM((2, GB, 128, C), jnp.float32),
            pltpu.VMEM((2, RB, C), jnp.float32),
            pltpu.SemaphoreType.DMA((2,)),
            pltpu.SemaphoreType.DMA((2,)),
            pltpu.SemaphoreType.DMA((2,)),
            pltpu.SemaphoreType.DMA,
        ],
    )(inference, gt128)
    return (soft, ohe3.reshape(B, C))


# speedup vs baseline: 1.0361x; 1.0361x over previous
import jax
import jax.numpy as jnp
from jax.experimental import pallas as pl
from jax.experimental.pallas import tpu as pltpu

B = 16384
C = 10
NB = 2
RB = B // NB
GB = 128 // NB


def _body(x_hbm, g_hbm, soft_hbm, ohe_hbm,
          ebuf, xbuf, gbuf, obuf, sbuf, xsem, osem, ssem, gsem):
    def xcopy(b, slot):
        return pltpu.make_async_copy(
            x_hbm.at[pl.ds(b * RB, RB), :], xbuf.at[slot], xsem.at[slot])

    def ocopy(b, slot):
        return pltpu.make_async_copy(
            obuf.at[slot], ohe_hbm.at[pl.ds(b * GB, GB)], osem.at[slot])

    def scopy(b, slot):
        return pltpu.make_async_copy(
            sbuf.at[slot], soft_hbm.at[pl.ds(b * RB, RB), :], ssem.at[slot])

    gcopy = pltpu.make_async_copy(g_hbm, gbuf, gsem)
    gcopy.start()
    xcopy(0, 0).start()
    xcopy(1, 1).start()
    gcopy.wait()

    def pass1(b, acc):
        slot = jax.lax.rem(b, 2)
        xcopy(b, slot).wait()
        e = jnp.exp(xbuf[slot])
        ebuf[pl.ds(b * RB, RB), :] = e

        @pl.when(b + 2 < NB)
        def _prefetch():
            xcopy(b + 2, slot).start()

        @pl.when(b >= 2)
        def _drain():
            ocopy(b - 2, slot).wait()

        g = gbuf[pl.ds(b * GB, GB), :]
        cls = jax.lax.broadcasted_iota(jnp.int32, (GB, 128, C), 2)
        obuf[slot] = (g[:, :, None] == cls).astype(jnp.float32)
        ocopy(b, slot).start()
        return acc + jnp.sum(e)

    total = jax.lax.fori_loop(0, NB, pass1, 0.0)
    inv = 1.0 / total

    def pass2(b, carry):
        slot = jax.lax.rem(b, 2)

        @pl.when(b >= 2)
        def _drain():
            scopy(b - 2, slot).wait()

        sbuf[slot] = ebuf[pl.ds(b * RB, RB), :] * inv
        scopy(b, slot).start()
        return carry

    jax.lax.fori_loop(0, NB, pass2, 0)

    ocopy(NB - 2, 0).wait()
    ocopy(NB - 1, 1).wait()
    scopy(NB - 2, 0).wait()
    scopy(NB - 1, 1).wait()


def kernel(inference, ground_truth):
    gt128 = ground_truth.astype(jnp.int32).reshape(128, 128)
    soft, ohe3 = pl.pallas_call(
        _body,
        in_specs=[
            pl.BlockSpec(memory_space=pltpu.MemorySpace.HBM),
            pl.BlockSpec(memory_space=pltpu.MemorySpace.HBM),
        ],
        out_specs=[
            pl.BlockSpec(memory_space=pltpu.MemorySpace.HBM),
            pl.BlockSpec(memory_space=pltpu.MemorySpace.HBM),
        ],
        out_shape=(
            jax.ShapeDtypeStruct((B, C), jnp.float32),
            jax.ShapeDtypeStruct((128, 128, C), jnp.float32),
        ),
        scratch_shapes=[
            pltpu.VMEM((B, C), jnp.float32),
            pltpu.VMEM((2, RB, C), jnp.float32),
            pltpu.VMEM((128, 128), jnp.int32),
            pltpu.VMEM((2, GB, 128, C), jnp.float32),
            pltpu.VMEM((2, RB, C), jnp.float32),
            pltpu.SemaphoreType.DMA((2,)),
            pltpu.SemaphoreType.DMA((2,)),
            pltpu.SemaphoreType.DMA((2,)),
            pltpu.SemaphoreType.DMA,
        ],
    )(inference, gt128)
    return (soft, ohe3.reshape(B, C))
